# baseline (speedup 1.0000x reference)
import functools
import math

import jax
import jax.numpy as jnp
from jax import lax
from jax.experimental import pallas as pl
from jax.experimental.pallas import tpu as pltpu
from jax.experimental.pallas import tpu_sc as plsc

D_MODEL = 64
SCALE = math.sqrt(D_MODEL)
CHUNK = 128


def kernel(tokens, table):
    b, s = tokens.shape
    v, d = table.shape
    n = b * s
    flat_tokens = tokens.reshape(n).astype(jnp.int32)

    info = plsc.get_sparse_core_info()
    num_workers = info.num_cores * info.num_subcores
    n_per_w = n // num_workers
    assert n % num_workers == 0 and n_per_w % CHUNK == 0
    steps = n_per_w // CHUNK

    mesh = plsc.VectorSubcoreMesh(core_axis_name="c", subcore_axis_name="s")

    @functools.partial(
        pl.kernel,
        mesh=mesh,
        out_type=jax.ShapeDtypeStruct((n, d), jnp.float32),
        scratch_types=[
            pltpu.VMEM((CHUNK,), jnp.int32),
            pltpu.VMEM((CHUNK, d), jnp.float32),
            pltpu.SemaphoreType.DMA,
        ],
        compiler_params=pltpu.CompilerParams(use_tc_tiling_on_sc=False),
    )
    def emb_kernel(tok_hbm, table_hbm, out_hbm, idx_v, rows_v, sem):
        wid = lax.axis_index("s") * info.num_cores + lax.axis_index("c")
        base = wid * n_per_w

        def step(g, carry):
            off = base + g * CHUNK
            pltpu.sync_copy(tok_hbm.at[pl.ds(off, CHUNK)], idx_v)
            pltpu.async_copy(table_hbm.at[idx_v], rows_v, sem).wait()

            def scale_row(r, carry2):
                for c in range(d // 16):
                    sl = pl.ds(c * 16, 16)
                    rows_v[r, sl] = rows_v[r, sl] * SCALE
                return carry2

            lax.fori_loop(0, CHUNK, scale_row, 0)
            pltpu.sync_copy(rows_v, out_hbm.at[pl.ds(off, CHUNK)])
            return carry

        lax.fori_loop(0, steps, step, 0)

    out = emb_kernel(flat_tokens, table)
    return out.reshape(b, s, d)


# speedup vs baseline: 1.2629x; 1.2629x over previous
import functools
import math

import jax
import jax.numpy as jnp
from jax import lax
from jax.experimental import pallas as pl
from jax.experimental.pallas import tpu as pltpu
from jax.experimental.pallas import tpu_sc as plsc

D_MODEL = 64
SCALE = math.sqrt(D_MODEL)
CHUNK = 128
NBUF = 4


def kernel(tokens, table):
    b, s = tokens.shape
    v, d = table.shape
    n = b * s
    tok2d = tokens.reshape(n // CHUNK, CHUNK).astype(jnp.int32)

    info = plsc.get_sparse_core_info()
    num_workers = info.num_cores * info.num_subcores
    n_per_w = n // num_workers
    steps = n_per_w // CHUNK
    nrounds = steps // NBUF
    assert n % num_workers == 0 and n_per_w % CHUNK == 0 and steps % NBUF == 0

    mesh = plsc.VectorSubcoreMesh(core_axis_name="c", subcore_axis_name="s")

    @functools.partial(
        pl.kernel,
        mesh=mesh,
        out_type=jax.ShapeDtypeStruct((n, d), jnp.float32),
        scratch_types=[
            pltpu.VMEM((steps, CHUNK), jnp.int32),
            pltpu.VMEM((NBUF, CHUNK, d), jnp.float32),
        ]
        + [pltpu.SemaphoreType.DMA] * (2 * NBUF),
        compiler_params=pltpu.CompilerParams(use_tc_tiling_on_sc=False),
    )
    def emb_kernel(tok_hbm, table_hbm, out_hbm, idx_all, rows, *sems):
        gsems = sems[:NBUF]
        osems = sems[NBUF:]
        wid = lax.axis_index("s") * info.num_cores + lax.axis_index("c")
        base = wid * n_per_w

        pltpu.sync_copy(tok_hbm.at[pl.ds(wid * steps, steps)], idx_all)

        def gather_start(g, bb):
            pltpu.async_copy(table_hbm.at[idx_all.at[g]], rows.at[bb], gsems[bb])

        def gather_wait(g, bb):
            pltpu.make_async_copy(
                table_hbm.at[idx_all.at[g]], rows.at[bb], gsems[bb]
            ).wait()

        def scatter_start(g, bb):
            off = base + g * CHUNK
            pltpu.async_copy(rows.at[bb], out_hbm.at[pl.ds(off, CHUNK)], osems[bb])

        def scatter_wait(g, bb):
            off = base + g * CHUNK
            pltpu.make_async_copy(
                rows.at[bb], out_hbm.at[pl.ds(off, CHUNK)], osems[bb]
            ).wait()

        def scale(bb):
            @plsc.parallel_loop(0, CHUNK, unroll=4)
            def _(r):
                for c in range(d // 16):
                    sl = pl.ds(c * 16, 16)
                    rows[bb, r, sl] = rows[bb, r, sl] * SCALE

        for bb in range(NBUF):
            gather_start(bb, bb)

        def round_body(t, carry):
            g0 = t * NBUF
            for bb in range(NBUF):
                g = g0 + bb
                gather_wait(g, bb)
                scale(bb)
                scatter_start(g, bb)

            @pl.when(t + 1 < nrounds)
            def _():
                for bb in range(NBUF):
                    g = g0 + bb
                    scatter_wait(g, bb)
                    gather_start(g + NBUF, bb)

            return carry

        lax.fori_loop(0, nrounds, round_body, 0)

        for bb in range(NBUF):
            scatter_wait((nrounds - 1) * NBUF + bb, bb)

    out = emb_kernel(tok2d, table)
    return out.reshape(b, s, d)
